# Initial kernel scaffold; baseline (speedup 1.0000x reference)
import jax
import jax.numpy as jnp
from jax.experimental import pallas as pl


def kernel(x, depth, emb0, emb1, depth_table):
    raise NotImplementedError("write your pallas kernel here")


# speedup vs baseline: 3.7460x; 3.7460x over previous
import functools

import jax
import jax.numpy as jnp
from jax import lax
from jax.experimental import pallas as pl
from jax.experimental.pallas import tpu as pltpu
from jax.experimental.pallas import tpu_sc as plsc

_EMB_DIM = 128
_D0 = 42
_D1 = 42
_DD = 44
_V0 = 98
_VD = 21
_MAX_DEPTH = 20
_N = 100000
_B = 512
_SUB = 128
_NFULL = _N // _B
_REM = _N - _NFULL * _B
_NW = 32


def kernel(x, depth, emb0, emb1, depth_table):
    x0 = x[:, 0].astype(jnp.int32)
    x1 = x[:, 1].astype(jnp.int32)
    dep = depth.astype(jnp.int32)
    e1p = jnp.pad(emb1, ((0, 0), (_D0, _EMB_DIM - _D0 - _D1)))
    e0f = emb0.reshape(-1)
    edf = depth_table.reshape(-1)

    mesh = plsc.VectorSubcoreMesh(core_axis_name="c", subcore_axis_name="s")

    @functools.partial(
        pl.kernel,
        mesh=mesh,
        out_type=jax.ShapeDtypeStruct((_N, _EMB_DIM), jnp.float32),
        scratch_types=[
            pltpu.VMEM((_B,), jnp.int32),
            pltpu.VMEM((_B,), jnp.int32),
            pltpu.VMEM((_B,), jnp.int32),
            pltpu.VMEM((_V0 * _D0,), jnp.float32),
            pltpu.VMEM((_VD * _DD,), jnp.float32),
            pltpu.VMEM((_B, _EMB_DIM), jnp.float32),
            pltpu.SemaphoreType.DMA,
        ],
    )
    def run(x0_hbm, x1_hbm, dep_hbm, e0_hbm, e1_hbm, ed_hbm, out_hbm,
            idx0, idx1, idxd, e0_res, ed_res, rows_v, sem):
        wid = lax.axis_index("s") * 2 + lax.axis_index("c")

        pltpu.sync_copy(e0_hbm, e0_res)
        pltpu.sync_copy(ed_hbm, ed_res)

        def do_chunk(base, nrows):
            pltpu.sync_copy(x0_hbm.at[pl.ds(base, nrows)], idx0.at[pl.ds(0, nrows)])
            pltpu.sync_copy(x1_hbm.at[pl.ds(base, nrows)], idx1.at[pl.ds(0, nrows)])
            pltpu.sync_copy(dep_hbm.at[pl.ds(base, nrows)], idxd.at[pl.ds(0, nrows)])

            copies = []
            nsub = nrows // _SUB
            for j in range(nsub + (1 if nrows % _SUB else 0)):
                cnt = _SUB if j < nsub else nrows % _SUB
                sl = pl.ds(j * _SUB, cnt)
                copies.append(pltpu.async_copy(e1_hbm.at[idx1.at[sl]], rows_v.at[sl], sem))
            for c in copies:
                c.wait()

            def fill(t, _):
                r0 = t * 16
                av = idx0[pl.ds(r0, 16)] * _D0
                dv = jnp.minimum(idxd[pl.ds(r0, 16)], _MAX_DEPTH) * _DD
                for l in range(16):
                    r = r0 + l
                    i0 = av[l]
                    idep = dv[l]
                    rows_v[r, pl.ds(0, 16)] = e0_res[pl.ds(i0, 16)]
                    rows_v[r, pl.ds(16, 16)] = e0_res[pl.ds(i0 + 16, 16)]
                    rows_v[r, pl.ds(_D0 - 16, 16)] = e0_res[pl.ds(i0 + _D0 - 16, 16)]
                    rows_v[r, pl.ds(_D0 + _D1, 16)] = ed_res[pl.ds(idep, 16)]
                    rows_v[r, pl.ds(_D0 + _D1 + 16, 16)] = ed_res[pl.ds(idep + 16, 16)]
                    rows_v[r, pl.ds(_EMB_DIM - 16, 16)] = ed_res[pl.ds(idep + _DD - 16, 16)]
                return 0
            lax.fori_loop(0, nrows // 16, fill, 0)

            pltpu.sync_copy(rows_v.at[pl.ds(0, nrows)], out_hbm.at[pl.ds(base, nrows)])

        nchunks = jnp.where(wid < (_NFULL % _NW), _NFULL // _NW + 1, _NFULL // _NW)

        def body(g, _):
            do_chunk((wid + g * _NW) * _B, _B)
            return 0
        lax.fori_loop(0, nchunks, body, 0)

        if _REM:
            @pl.when(wid == _NW - 1)
            def _():
                do_chunk(_NFULL * _B, _REM)

    return run(x0, x1, dep, e0f, e1p, edf)
